# baseline (speedup 1.0000x reference)
import jax
import jax.numpy as jnp
from jax import lax
from jax.experimental import pallas as pl
from jax.experimental.pallas import tpu as pltpu
from jax.experimental.pallas import tpu_sc as plsc

L = 16
NC = 2
NS = 16
NW = NC * NS
B = 16384
D = 32
BW = B // NW
PITCH = L + 1


def _sc_body(head_hbm, rel_hbm, tail_hbm, tab_hbm, out_hbm,
             hidx, ridx, tidx, h_v, r_v, t_v, out_v, sem):
    wid = lax.axis_index("s") * NC + lax.axis_index("c")
    base = wid * BW

    pltpu.sync_copy(head_hbm.at[pl.ds(base, BW)], hidx)
    pltpu.sync_copy(rel_hbm.at[pl.ds(base, BW)], ridx)
    pltpu.sync_copy(tail_hbm.at[pl.ds(base, BW)], tidx)

    ch = pltpu.async_copy(tab_hbm.at[hidx], h_v, sem)
    cr = pltpu.async_copy(tab_hbm.at[ridx], r_v, sem)
    ct = pltpu.async_copy(tab_hbm.at[tidx], t_v, sem)
    ch.wait()
    cr.wait()
    ct.wait()

    mask0 = lax.iota(jnp.int32, L) == 0

    def row_body(i, carry):
        for u in range(4):
            r = i * 4 + u
            d0 = h_v[r, pl.ds(0, L)] + r_v[r, pl.ds(0, L)] - t_v[r, pl.ds(0, L)]
            d1 = h_v[r, pl.ds(L, L)] + r_v[r, pl.ds(L, L)] - t_v[r, pl.ds(L, L)]
            s = d0 * d0 + d1 * d1
            total = lax.broadcast(jnp.sum(s, axis=0), (L,))
            plsc.store_compressed(out_v.at[pl.ds(r, L)], total, mask=mask0)
        return carry

    lax.fori_loop(0, BW // 4, row_body, 0)

    pltpu.sync_copy(out_v.at[pl.ds(0, BW)], out_hbm.at[pl.ds(base, BW)])


def kernel(head_idx, rel_idx, tail_idx, entities_emb, labels_emb):
    table = jnp.concatenate([entities_emb, labels_emb], axis=0)
    rel_off = rel_idx + entities_emb.shape[0]
    mesh = plsc.VectorSubcoreMesh(core_axis_name="c", subcore_axis_name="s")
    k = pl.kernel(
        _sc_body,
        out_type=jax.ShapeDtypeStruct((B,), jnp.float32),
        mesh=mesh,
        compiler_params=pltpu.CompilerParams(
            needs_layout_passes=False, use_tc_tiling_on_sc=False),
        scratch_types=[
            pltpu.VMEM((BW,), jnp.int32),
            pltpu.VMEM((BW,), jnp.int32),
            pltpu.VMEM((BW,), jnp.int32),
            pltpu.VMEM((BW, D), jnp.float32),
            pltpu.VMEM((BW, D), jnp.float32),
            pltpu.VMEM((BW, D), jnp.float32),
            pltpu.VMEM((BW + L,), jnp.float32),
            pltpu.SemaphoreType.DMA,
        ],
    )
    return k(head_idx, rel_off, tail_idx, table)


# speedup vs baseline: 1.0446x; 1.0446x over previous
import jax
import jax.numpy as jnp
from jax import lax
from jax.experimental import pallas as pl
from jax.experimental.pallas import tpu as pltpu
from jax.experimental.pallas import tpu_sc as plsc

L = 16
NC = 2
NS = 16
NW = NC * NS
B = 16384
D = 32
BW = B // NW
PITCH = L + 1


def _sc_body(head_hbm, rel_hbm, tail_hbm, ent_hbm, lab_hbm, out_hbm,
             hidx, ridx, tidx, h_v, r_v, t_v, out_v, sem):
    wid = lax.axis_index("s") * NC + lax.axis_index("c")
    base = wid * BW

    pltpu.sync_copy(head_hbm.at[pl.ds(base, BW)], hidx)
    pltpu.sync_copy(rel_hbm.at[pl.ds(base, BW)], ridx)
    pltpu.sync_copy(tail_hbm.at[pl.ds(base, BW)], tidx)

    ch = pltpu.async_copy(ent_hbm.at[hidx], h_v, sem)
    cr = pltpu.async_copy(lab_hbm.at[ridx], r_v, sem)
    ct = pltpu.async_copy(ent_hbm.at[tidx], t_v, sem)
    ch.wait()
    cr.wait()
    ct.wait()

    mask0 = lax.iota(jnp.int32, L) == 0

    def row_body(i, carry):
        for u in range(4):
            r = i * 4 + u
            d0 = h_v[r, pl.ds(0, L)] + r_v[r, pl.ds(0, L)] - t_v[r, pl.ds(0, L)]
            d1 = h_v[r, pl.ds(L, L)] + r_v[r, pl.ds(L, L)] - t_v[r, pl.ds(L, L)]
            s = d0 * d0 + d1 * d1
            total = lax.broadcast(jnp.sum(s, axis=0), (L,))
            plsc.store_compressed(out_v.at[pl.ds(r, L)], total, mask=mask0)
        return carry

    lax.fori_loop(0, BW // 4, row_body, 0)

    pltpu.sync_copy(out_v.at[pl.ds(0, BW)], out_hbm.at[pl.ds(base, BW)])


def kernel(head_idx, rel_idx, tail_idx, entities_emb, labels_emb):
    mesh = plsc.VectorSubcoreMesh(core_axis_name="c", subcore_axis_name="s")
    k = pl.kernel(
        _sc_body,
        out_type=jax.ShapeDtypeStruct((B,), jnp.float32),
        mesh=mesh,
        compiler_params=pltpu.CompilerParams(
            needs_layout_passes=False, use_tc_tiling_on_sc=False),
        scratch_types=[
            pltpu.VMEM((BW,), jnp.int32),
            pltpu.VMEM((BW,), jnp.int32),
            pltpu.VMEM((BW,), jnp.int32),
            pltpu.VMEM((BW, D), jnp.float32),
            pltpu.VMEM((BW, D), jnp.float32),
            pltpu.VMEM((BW, D), jnp.float32),
            pltpu.VMEM((BW + L,), jnp.float32),
            pltpu.SemaphoreType.DMA,
        ],
    )
    return k(head_idx, rel_idx, tail_idx, entities_emb, labels_emb)
